# Initial kernel scaffold; baseline (speedup 1.0000x reference)
import jax
import jax.numpy as jnp
from jax.experimental import pallas as pl


def kernel(input, row_idx, col_idx, enc_W1, enc_b1, enc_W2, enc_b2, enc_W3, enc_b3, dec_W1, dec_b1, dec_W2, dec_b2, dec_W3, dec_b3):
    raise NotImplementedError("write your pallas kernel here")


# speedup vs baseline: 2.0944x; 2.0944x over previous
import jax
import jax.numpy as jnp
from jax import lax
from jax.experimental import pallas as pl
from jax.experimental.pallas import tpu as pltpu
from jax.experimental.pallas import tpu_sc as plsc

NNZ = 160000
NSEG = 10000
NSEG_PAD = 10240
G = 128
NGROUPS = NNZ // G
NSUB = 16
NCORES = 2
ROWS_PER_TILE = NSEG_PAD // NSUB

_MESH = dict(core_axis_name="c", subcore_axis_name="s")


def _chunks(w):
    return [w]


def _sc_scatter(xs, ridx, cidx):
    ws = [x.shape[1] for x in xs]
    nch = len(xs)

    def body(*refs):
        x_hbms = refs[:nch]
        ridx_hbm, cidx_hbm = refs[nch], refs[nch + 1]
        rowsum_hbms = refs[nch + 2:2 * nch + 2]
        colsum_hbms = refs[2 * nch + 2:3 * nch + 2]
        scratch = refs[3 * nch + 2:]
        tables = scratch[:nch]
        xbufs = scratch[nch:2 * nch]
        idxbuf = scratch[2 * nch]

        cid = lax.axis_index("c")
        sid = lax.axis_index("s")
        lo = (NGROUPS * sid) // NSUB
        hi = (NGROUPS * (sid + 1)) // NSUB
        r0 = sid * ROWS_PER_TILE

        def zrow(r, carry):
            for c in range(nch):
                for k in range(ws[c] // 16):
                    xbufs[c][r, pl.ds(k * 16, 16)] = jnp.zeros(
                        (16,), jnp.float32)
            return carry
        lax.fori_loop(0, G, zrow, 0)
        for c in range(nch):
            for piece in range(ROWS_PER_TILE // G):
                pltpu.sync_copy(
                    xbufs[c], tables[c].at[pl.ds(r0 + piece * G, G)])
        plsc.subcore_barrier()

        def do_scatter(idx_hbm):
            def step(g, carry):
                pltpu.sync_copy(idx_hbm.at[pl.ds(g * G, G)], idxbuf)
                for c in range(nch):
                    pltpu.sync_copy(x_hbms[c].at[pl.ds(g * G, G)], xbufs[c])
                    pltpu.sync_copy(xbufs[c], tables[c].at[idxbuf], add=True)
                return carry
            lax.fori_loop(lo, hi, step, 0)

        @pl.when(cid == 0)
        def _():
            do_scatter(ridx_hbm)

        @pl.when(cid == 1)
        def _():
            do_scatter(cidx_hbm)

        plsc.subcore_barrier()

        @pl.when(cid == 0)
        def _():
            for c in range(nch):
                pltpu.sync_copy(tables[c].at[pl.ds(r0, ROWS_PER_TILE)],
                                rowsum_hbms[c].at[pl.ds(r0, ROWS_PER_TILE)])

        @pl.when(cid == 1)
        def _():
            for c in range(nch):
                pltpu.sync_copy(tables[c].at[pl.ds(r0, ROWS_PER_TILE)],
                                colsum_hbms[c].at[pl.ds(r0, ROWS_PER_TILE)])

    tab_t = [jax.ShapeDtypeStruct((NSEG_PAD, w), jnp.float32) for w in ws]
    f = pl.kernel(
        body,
        out_type=tuple(tab_t) + tuple(tab_t),
        mesh=plsc.VectorSubcoreMesh(**_MESH),
        compiler_params=pltpu.CompilerParams(use_tc_tiling_on_sc=False),
        scratch_types=(
            [pltpu.VMEM_SHARED((NSEG_PAD, w), jnp.float32) for w in ws]
            + [pltpu.VMEM((G, w), jnp.float32) for w in ws]
            + [pltpu.VMEM((G,), jnp.int32)]
        ),
    )
    out = f(*xs, ridx, cidx)
    return list(out[:nch]), list(out[nch:])


def _sc_gather_combine(rowtabs, coltabs, xws, ridx, cidx, *, lrelu):
    ws = [t.shape[1] for t in rowtabs]
    nch = len(rowtabs)

    def body(*refs):
        rowtab_hbms = refs[:nch]
        coltab_hbms = refs[nch:2 * nch]
        xw_hbms = refs[2 * nch:3 * nch]
        ridx_hbm, cidx_hbm = refs[3 * nch], refs[3 * nch + 1]
        out_hbms = refs[3 * nch + 2:4 * nch + 2]
        scratch = refs[4 * nch + 2:]
        rbufs = scratch[:nch]
        cbufs = scratch[nch:2 * nch]
        xbufs = scratch[2 * nch:3 * nch]
        obufs = scratch[3 * nch:4 * nch]
        ridxbuf, cidxbuf, sem = scratch[4 * nch:]

        cid = lax.axis_index("c")
        sid = lax.axis_index("s")
        wid = cid * NSUB + sid
        lo = (NGROUPS * wid) // (NCORES * NSUB)
        hi = (NGROUPS * (wid + 1)) // (NCORES * NSUB)

        def step(g, carry):
            pltpu.sync_copy(ridx_hbm.at[pl.ds(g * G, G)], ridxbuf)
            pltpu.sync_copy(cidx_hbm.at[pl.ds(g * G, G)], cidxbuf)
            for c in range(nch):
                pltpu.async_copy(rowtab_hbms[c].at[ridxbuf], rbufs[c],
                                 sem).wait()
                pltpu.async_copy(coltab_hbms[c].at[cidxbuf], cbufs[c],
                                 sem).wait()
                pltpu.sync_copy(xw_hbms[c].at[pl.ds(g * G, G)], xbufs[c])

            def vrow(r, carry2):
                for c in range(nch):
                    for k in range(ws[c] // 16):
                        sl = pl.ds(k * 16, 16)
                        a = (xbufs[c][r, sl] + rbufs[c][r, sl]
                             + cbufs[c][r, sl])
                        if lrelu:
                            a = jnp.maximum(a, a * jnp.float32(0.01))
                        obufs[c][r, sl] = a
                return carry2
            lax.fori_loop(0, G, vrow, 0)
            for c in range(nch):
                pltpu.sync_copy(obufs[c], out_hbms[c].at[pl.ds(g * G, G)])
            return carry
        lax.fori_loop(lo, hi, step, 0)

    f = pl.kernel(
        body,
        out_type=tuple(jax.ShapeDtypeStruct((NNZ, w), jnp.float32)
                       for w in ws),
        mesh=plsc.VectorSubcoreMesh(**_MESH),
        compiler_params=pltpu.CompilerParams(use_tc_tiling_on_sc=False),
        scratch_types=(
            [pltpu.VMEM((G, w), jnp.float32) for w in ws] * 4
            + [pltpu.VMEM((G,), jnp.int32),
               pltpu.VMEM((G,), jnp.int32),
               pltpu.SemaphoreType.DMA]
        ),
    )
    out = f(*rowtabs, *coltabs, *xws, ridx, cidx)
    return list(out)


def _sc_gather_concat(rowtab, coltab, ridx, cidx):
    o = rowtab.shape[1]

    def body(rowtab_hbm, coltab_hbm, ridx_hbm, cidx_hbm, out_hbm,
             rbuf, cbuf, obuf, ridxbuf, cidxbuf, sem):
        cid = lax.axis_index("c")
        sid = lax.axis_index("s")
        wid = cid * NSUB + sid
        lo = (NGROUPS * wid) // (NCORES * NSUB)
        hi = (NGROUPS * (wid + 1)) // (NCORES * NSUB)

        def step(g, carry):
            pltpu.sync_copy(ridx_hbm.at[pl.ds(g * G, G)], ridxbuf)
            pltpu.sync_copy(cidx_hbm.at[pl.ds(g * G, G)], cidxbuf)
            pltpu.async_copy(rowtab_hbm.at[ridxbuf], rbuf, sem).wait()
            pltpu.async_copy(coltab_hbm.at[cidxbuf], cbuf, sem).wait()

            def vrow(r, carry2):
                for k in range(o // 16):
                    obuf[r, pl.ds(k * 16, 16)] = rbuf[r, pl.ds(k * 16, 16)]
                    obuf[r, pl.ds(o + k * 16, 16)] = cbuf[r, pl.ds(k * 16, 16)]
                return carry2
            lax.fori_loop(0, G, vrow, 0)
            pltpu.sync_copy(obuf, out_hbm.at[pl.ds(g * G, G)])
            return carry
        lax.fori_loop(lo, hi, step, 0)

    f = pl.kernel(
        body,
        out_type=jax.ShapeDtypeStruct((NNZ, 2 * o), jnp.float32),
        mesh=plsc.VectorSubcoreMesh(**_MESH),
        compiler_params=pltpu.CompilerParams(use_tc_tiling_on_sc=False),
        scratch_types=[
            pltpu.VMEM((G, o), jnp.float32),
            pltpu.VMEM((G, o), jnp.float32),
            pltpu.VMEM((G, 2 * o), jnp.float32),
            pltpu.VMEM((G,), jnp.int32),
            pltpu.VMEM((G,), jnp.int32),
            pltpu.SemaphoreType.DMA,
        ],
    )
    return f(rowtab, coltab, ridx, cidx)


def _tc_edges(xs, WxTs, blk=2000):
    n = xs[0].shape[0]
    ws = [x.shape[1] for x in xs]
    os_ = [w.shape[1] for w in WxTs]
    nin, nout = len(xs), len(WxTs)
    grid = n // blk

    def body(*refs):
        x_refs = refs[:nin]
        w_refs = refs[nin:nin + nout]
        xw_refs = refs[nin + nout:nin + 2 * nout]
        gs_refs = refs[nin + 2 * nout:]
        xb = jnp.concatenate([r[...] for r in x_refs], axis=1)
        for c in range(nout):
            xw_refs[c][...] = jnp.dot(xb, w_refs[c][...],
                                      preferred_element_type=jnp.float32)

        @pl.when(pl.program_id(0) == 0)
        def _():
            for c in range(nin):
                gs_refs[c][...] = jnp.zeros_like(gs_refs[c])
        for c in range(nin):
            gs_refs[c][...] += jnp.sum(x_refs[c][...], axis=0, keepdims=True)

    wtot = sum(ws)
    return pl.pallas_call(
        body,
        grid=(grid,),
        in_specs=([pl.BlockSpec((blk, w), lambda i: (i, 0)) for w in ws]
                  + [pl.BlockSpec((wtot, o), lambda i: (0, 0)) for o in os_]),
        out_specs=([pl.BlockSpec((blk, o), lambda i: (i, 0)) for o in os_]
                   + [pl.BlockSpec((1, w), lambda i: (0, 0)) for w in ws]),
        out_shape=([jax.ShapeDtypeStruct((n, o), jnp.float32) for o in os_]
                   + [jax.ShapeDtypeStruct((1, w), jnp.float32) for w in ws]),
    )(*xs, *WxTs)


def _tc_tables(rowsums, colsums, rcnt, ccnt, gsums,
               WrTs, WcTs, WbTs, bs, blk=2048):
    ws = [x.shape[1] for x in rowsums]
    os_ = [w.shape[1] for w in WrTs]
    nin, nout = len(rowsums), len(WrTs)
    grid = NSEG_PAD // blk
    wtot = sum(ws)

    def body(*refs):
        i = 0
        rs_refs = refs[i:i + nin]; i += nin
        cs_refs = refs[i:i + nin]; i += nin
        rc_ref = refs[i]; i += 1
        cc_ref = refs[i]; i += 1
        gs_refs = refs[i:i + nin]; i += nin
        wr_refs = refs[i:i + nout]; i += nout
        wc_refs = refs[i:i + nout]; i += nout
        wb_refs = refs[i:i + nout]; i += nout
        b_refs = refs[i:i + nout]; i += nout
        rowtab_refs = refs[i:i + nout]; i += nout
        coltab_refs = refs[i:i + nout]

        gs = jnp.concatenate([r[...] for r in gs_refs], axis=1)
        inv_r = 1.0 / jnp.maximum(rc_ref[...], 1.0)
        inv_c = 1.0 / jnp.maximum(cc_ref[...], 1.0)
        rmean = jnp.concatenate([r[...] for r in rs_refs], axis=1) * inv_r
        cmean = jnp.concatenate([r[...] for r in cs_refs], axis=1) * inv_c
        for c in range(nout):
            const = (jnp.dot(gs * jnp.float32(1.0 / NNZ), wb_refs[c][...],
                             preferred_element_type=jnp.float32)
                     + b_refs[c][...])
            rowtab_refs[c][...] = jnp.dot(
                rmean, wr_refs[c][...],
                preferred_element_type=jnp.float32) + const
            coltab_refs[c][...] = jnp.dot(
                cmean, wc_refs[c][...],
                preferred_element_type=jnp.float32)

    tab_shape = [jax.ShapeDtypeStruct((NSEG_PAD, o), jnp.float32)
                 for o in os_]
    out = pl.pallas_call(
        body,
        grid=(grid,),
        in_specs=([pl.BlockSpec((blk, w), lambda i: (i, 0)) for w in ws] * 2
                  + [pl.BlockSpec((blk, 1), lambda i: (i, 0))] * 2
                  + [pl.BlockSpec((1, w), lambda i: (0, 0)) for w in ws]
                  + [pl.BlockSpec((wtot, o), lambda i: (0, 0))
                     for o in os_] * 3
                  + [pl.BlockSpec((1, o), lambda i: (0, 0)) for o in os_]),
        out_specs=[pl.BlockSpec((blk, o), lambda i: (i, 0))
                   for o in os_] * 2,
        out_shape=tab_shape * 2,
    )(*rowsums, *colsums, rcnt, ccnt, *gsums, *WrTs, *WcTs, *WbTs, *bs)
    return out[:nout], out[nout:]


def _tc_means(rowsum, colsum, rcnt, ccnt, blk=2048):
    w = rowsum.shape[1]
    grid = NSEG_PAD // blk

    def body(rs_ref, cs_ref, rc_ref, cc_ref, rm_ref, cm_ref):
        rm_ref[...] = rs_ref[...] / jnp.maximum(rc_ref[...], 1.0)
        cm_ref[...] = cs_ref[...] / jnp.maximum(cc_ref[...], 1.0)

    return pl.pallas_call(
        body,
        grid=(grid,),
        in_specs=[pl.BlockSpec((blk, w), lambda i: (i, 0)),
                  pl.BlockSpec((blk, w), lambda i: (i, 0)),
                  pl.BlockSpec((blk, 1), lambda i: (i, 0)),
                  pl.BlockSpec((blk, 1), lambda i: (i, 0))],
        out_specs=[pl.BlockSpec((blk, w), lambda i: (i, 0)),
                   pl.BlockSpec((blk, w), lambda i: (i, 0))],
        out_shape=[jax.ShapeDtypeStruct((NSEG_PAD, w), jnp.float32),
                   jax.ShapeDtypeStruct((NSEG_PAD, w), jnp.float32)],
    )(rowsum, colsum, rcnt, ccnt)


def _prep_weights(W, b, d, dpad, opad):
    o = W.shape[0]
    ochunks = _chunks(opad)
    factors = []
    for j in range(4):
        Wj = W[:, j * d:(j + 1) * d]
        Wj = jnp.pad(Wj, ((0, opad - o), (0, dpad - d))).T
        col = []
        off = 0
        for oc in ochunks:
            col.append(Wj[:, off:off + oc])
            off += oc
        factors.append(col)
    bpad = jnp.pad(b, (0, opad - o)).reshape(1, opad)
    bcol = []
    off = 0
    for oc in ochunks:
        bcol.append(bpad[:, off:off + oc])
        off += oc
    return factors[0], factors[1], factors[2], factors[3], bcol


def _layer(xs, ridx, cidx, rcnt, ccnt, Wparts, *, lrelu):
    WxTs, WrTs, WcTs, WbTs, bs = Wparts
    rowsums, colsums = _sc_scatter(xs, ridx, cidx)
    ed = _tc_edges(xs, WxTs)
    xws, gsums = ed[:len(WxTs)], ed[len(WxTs):]
    rowtabs, coltabs = _tc_tables(rowsums, colsums, rcnt, ccnt, gsums,
                                  WrTs, WcTs, WbTs, bs)
    return _sc_gather_combine(rowtabs, coltabs, xws, ridx, cidx, lrelu=lrelu)


def kernel(input, row_idx, col_idx,
           enc_W1, enc_b1, enc_W2, enc_b2, enc_W3, enc_b3,
           dec_W1, dec_b1, dec_W2, dec_b2, dec_W3, dec_b3):
    x0 = jnp.pad(input, ((0, 0), (0, 11))).at[:, 5].set(1.0)

    we1 = _prep_weights(enc_W1, enc_b1, 5, 16, 160)
    we2 = _prep_weights(enc_W2, enc_b2, 150, 160, 160)
    we3 = _prep_weights(enc_W3, enc_b3, 150, 160, 32)
    wd1 = _prep_weights(dec_W1, dec_b1, 64, 64, 160)
    wd2 = _prep_weights(dec_W2, dec_b2, 150, 160, 160)
    wd3 = _prep_weights(dec_W3, dec_b3, 150, 160, 16)

    rowsums1, colsums1 = _sc_scatter([x0], row_idx, col_idx)
    rcnt = rowsums1[0][:, 5:6]
    ccnt = colsums1[0][:, 5:6]
    ed1 = _tc_edges([x0], we1[0])
    xws1, gsums1 = ed1[:len(we1[0])], ed1[len(we1[0]):]
    rowtabs1, coltabs1 = _tc_tables(rowsums1, colsums1, rcnt, ccnt, gsums1,
                                    we1[1], we1[2], we1[3], we1[4])
    h = _sc_gather_combine(rowtabs1, coltabs1, xws1, row_idx, col_idx,
                           lrelu=True)

    h = _layer(h, row_idx, col_idx, rcnt, ccnt, we2, lrelu=True)
    encoded = _layer(h, row_idx, col_idx, rcnt, ccnt, we3, lrelu=False)

    prowsums, pcolsums = _sc_scatter(encoded, row_idx, col_idx)
    rowmean, colmean = _tc_means(prowsums[0], pcolsums[0], rcnt, ccnt)
    emb = _sc_gather_concat(rowmean, colmean, row_idx, col_idx)

    h = _layer([emb], row_idx, col_idx, rcnt, ccnt, wd1, lrelu=True)
    h = _layer(h, row_idx, col_idx, rcnt, ccnt, wd2, lrelu=True)
    out = _layer(h, row_idx, col_idx, rcnt, ccnt, wd3, lrelu=False)

    return out[0][:, :5]
